# baseline (speedup 1.0000x reference)
import functools

import jax
import jax.numpy as jnp
from jax import lax
from jax.experimental import pallas as pl
from jax.experimental.pallas import tpu as pltpu
from jax.experimental.pallas import tpu_sc as plsc

N = 4096
K = 32
EPS = 1e-10
BR = 512
NB = N // BR
NW = 32
BW_SC = N // NW


def _sc_gather(l1t_hbm, l0t_hbm, idx_hbm, out1_hbm, out2_hbm,
               idx_v, rows1_v, rows2_v, sem1, sem2):
    wid = lax.axis_index("s") * 2 + lax.axis_index("c")
    base = wid * BW_SC
    pltpu.sync_copy(idx_hbm.at[pl.ds(base, BW_SC)], idx_v)
    c1 = pltpu.async_copy(l1t_hbm.at[idx_v], rows1_v, sem1)
    c2 = pltpu.async_copy(l0t_hbm.at[idx_v], rows2_v, sem2)
    c1.wait()
    c2.wait()
    pltpu.sync_copy(rows1_v, out1_hbm.at[pl.ds(base, BW_SC)])
    pltpu.sync_copy(rows2_v, out2_hbm.at[pl.ds(base, BW_SC)])


def _gather_logprob_rows(l1t, l0t, idx):

    mesh = plsc.VectorSubcoreMesh(core_axis_name="c", subcore_axis_name="s")
    run = functools.partial(
        pl.kernel,
        mesh=mesh,
        out_type=[
            jax.ShapeDtypeStruct((N, 128), jnp.float32),
            jax.ShapeDtypeStruct((N, 128), jnp.float32),
        ],
        scratch_types=[
            pltpu.VMEM((BW_SC,), jnp.int32),
            pltpu.VMEM((BW_SC, 128), jnp.float32),
            pltpu.VMEM((BW_SC, 128), jnp.float32),
            pltpu.SemaphoreType.DMA,
            pltpu.SemaphoreType.DMA,
        ],
    )(_sc_gather)
    return run(l1t, l0t, idx)


def _em_body(l1z_ref, l0z_ref, adj_ref,
             resp_ref, asg_ref, p_ref, ll_ref,
             abf_scr, zn_scr, e_scr, ebf_scr,
             m_scr, nrow_scr, ncol_scr):
    i = pl.program_id(0)

    @pl.when(i < NB)
    def _estep():
        a = adj_ref[...]
        abf_scr[pl.ds(i * BR, BR), :] = a.astype(jnp.float8_e4m3fn)
        l1z = l1z_ref[:, pl.ds(0, K)]
        l0z = l0z_ref[:, pl.ds(0, K)]
        logp = (jnp.dot(a, l1z, preferred_element_type=jnp.float32)
                + jnp.dot(1.0 - a, l0z, preferred_element_type=jnp.float32))
        logp = logp - l0z_ref[pl.ds(i * BR, BR), pl.ds(0, K)]
        m = jnp.max(logp, axis=1, keepdims=True)
        e = jnp.exp(logp - m)
        r = e / jnp.sum(e, axis=1, keepdims=True)
        resp_ref[...] = r
        mx = jnp.max(r, axis=1, keepdims=True)
        kio = jax.lax.broadcasted_iota(jnp.int32, (BR, K), 1)
        asg = jnp.min(jnp.where(r == mx, kio, K), axis=1, keepdims=True)
        asg_ref[...] = asg
        zn_scr[pl.ds(i * BR, BR), :] = asg

    @pl.when(i == NB)
    def _init_m():
        kio = jax.lax.broadcasted_iota(jnp.int32, (N, K), 1)
        e_full = (zn_scr[...] == kio).astype(jnp.float32)
        e_scr[...] = e_full
        ebf_scr[...] = e_full.astype(jnp.float8_e4m3fn)
        nrow_scr[...] = jnp.sum(e_full, axis=0, keepdims=True)
        ones = jnp.ones((N, 1), jnp.float32)
        ncol_scr[...] = jax.lax.dot_general(
            e_full, ones, (((0,), (0,)), ((), ())),
            preferred_element_type=jnp.float32)
        m_scr[...] = jnp.zeros((K, K), jnp.float32)

    @pl.when(i >= NB)
    def _mstep():
        j = i - NB
        ab = abf_scr[pl.ds(j * BR, BR), :]
        r = jnp.dot(ab, ebf_scr[...], preferred_element_type=jnp.float32)
        e_blk = e_scr[pl.ds(j * BR, BR), :]
        m_scr[...] += jax.lax.dot_general(
            e_blk, r, (((0,), (0,)), ((), ())),
            preferred_element_type=jnp.float32)

    @pl.when(i == 2 * NB - 1)
    def _final():
        m = m_scr[...]
        nrow = nrow_scr[...]
        ncol = ncol_scr[...]
        ri = jax.lax.broadcasted_iota(jnp.int32, (K, K), 0)
        ci = jax.lax.broadcasted_iota(jnp.int32, (K, K), 1)
        eye = ri == ci
        counts_raw = ncol * nrow - jnp.where(eye, nrow, 0.0)
        counts = jnp.where(counts_raw == 0.0, 1.0, counts_raw)
        p = m / counts
        l1p = jnp.log(p + EPS)
        l0p = jnp.log(1.0 - p + EPS)
        ll = 0.5 * jnp.sum(m * l1p + (counts_raw - m) * l0p)
        p_ref[...] = p
        ll_ref[...] = jnp.full((1, 1), ll, jnp.float32)


def kernel(adj_matrix, block_assignments, block_probs):
    idx = block_assignments.astype(jnp.int32)
    l1t = jnp.log(block_probs + EPS).T
    l0t = jnp.log(1.0 - block_probs + EPS).T
    l1t_p = jnp.zeros((K, 128), jnp.float32).at[:, :K].set(l1t)
    l0t_p = jnp.zeros((K, 128), jnp.float32).at[:, :K].set(l0t)
    l1z, l0z = _gather_logprob_rows(l1t_p, l0t_p, idx)

    last = NB - 1
    resp, asg, p, ll = pl.pallas_call(
        _em_body,
        grid=(2 * NB,),
        in_specs=[
            pl.BlockSpec((N, 128), lambda i: (0, 0)),
            pl.BlockSpec((N, 128), lambda i: (0, 0)),
            pl.BlockSpec((BR, N), lambda i: (jnp.minimum(i, last), 0)),
        ],
        out_specs=[
            pl.BlockSpec((BR, K), lambda i: (jnp.minimum(i, last), 0)),
            pl.BlockSpec((BR, 1), lambda i: (jnp.minimum(i, last), 0)),
            pl.BlockSpec((K, K), lambda i: (0, 0)),
            pl.BlockSpec((1, 1), lambda i: (0, 0)),
        ],
        out_shape=[
            jax.ShapeDtypeStruct((N, K), jnp.float32),
            jax.ShapeDtypeStruct((N, 1), jnp.int32),
            jax.ShapeDtypeStruct((K, K), jnp.float32),
            jax.ShapeDtypeStruct((1, 1), jnp.float32),
        ],
        scratch_shapes=[
            pltpu.VMEM((N, N), jnp.float8_e4m3fn),
            pltpu.VMEM((N, 1), jnp.int32),
            pltpu.VMEM((N, K), jnp.float32),
            pltpu.VMEM((N, K), jnp.float8_e4m3fn),
            pltpu.VMEM((K, K), jnp.float32),
            pltpu.VMEM((1, K), jnp.float32),
            pltpu.VMEM((K, 1), jnp.float32),
        ],
    )(l1z, l0z, adj_matrix)

    return resp, asg.reshape(N), p, ll.reshape(())


# speedup vs baseline: 1.4979x; 1.4979x over previous
import jax
import jax.numpy as jnp
from jax.experimental import pallas as pl
from jax.experimental.pallas import tpu as pltpu

N = 4096
K = 32
EPS = 1e-10
BR = 512
HR = BR // 2
NB = N // BR
PREC = jax.lax.Precision.HIGHEST


def _em_body(z_ref, bp_ref, adj_ref, adj2_ref,
             resp_ref, asg_ref, p_ref, ll_ref,
             l1z_scr, l0z_scr, abf_scr, zn_scr, e_scr, ebf_scr,
             m_scr, nrow_scr, ncol_scr):
    i = pl.program_id(0)

    @pl.when(i == 0)
    def _init_e():
        bp = bp_ref[...]
        l1t = jnp.log(bp + EPS).T
        l0t = jnp.log(1.0 - bp + EPS).T
        kio = jax.lax.broadcasted_iota(jnp.int32, (N, K), 1)
        onehot = (z_ref[...] == kio).astype(jnp.float32)
        l1z_scr[...] = jnp.dot(onehot, l1t,
                               preferred_element_type=jnp.float32, precision=PREC)
        l0z_scr[...] = jnp.dot(onehot, l0t,
                               preferred_element_type=jnp.float32, precision=PREC)

    @pl.when(i < NB)
    def _estep():
        for h, a in ((0, adj_ref[...]), (1, adj2_ref[...])):
            base = i * BR + h * HR
            abf_scr[pl.ds(base, HR), :] = a.astype(jnp.float8_e4m3fn)
            logp = (jnp.dot(a, l1z_scr[...], preferred_element_type=jnp.float32)
                    + jnp.dot(1.0 - a, l0z_scr[...],
                              preferred_element_type=jnp.float32))
            logp = logp - l0z_scr[pl.ds(base, HR), :]
            m = jnp.max(logp, axis=1, keepdims=True)
            e = jnp.exp(logp - m)
            r = e / jnp.sum(e, axis=1, keepdims=True)
            resp_ref[pl.ds(h * HR, HR), :] = r
            mx = jnp.max(r, axis=1, keepdims=True)
            kio = jax.lax.broadcasted_iota(jnp.int32, (HR, K), 1)
            asg = jnp.min(jnp.where(r == mx, kio, K), axis=1, keepdims=True)
            asg_ref[pl.ds(h * HR, HR), :] = asg
            zn_scr[pl.ds(base, HR), :] = asg

    @pl.when(i == NB)
    def _init_m():
        kio = jax.lax.broadcasted_iota(jnp.int32, (N, K), 1)
        e_full = (zn_scr[...] == kio).astype(jnp.float32)
        e_scr[...] = e_full
        ebf_scr[...] = e_full.astype(jnp.float8_e4m3fn)
        nrow_scr[...] = jnp.sum(e_full, axis=0, keepdims=True)
        ones = jnp.ones((N, 1), jnp.float32)
        ncol_scr[...] = jax.lax.dot_general(
            e_full, ones, (((0,), (0,)), ((), ())),
            preferred_element_type=jnp.float32)
        m_scr[...] = jnp.zeros((K, K), jnp.float32)

    @pl.when(i >= NB)
    def _mstep():
        j = i - NB
        ab = abf_scr[pl.ds(j * BR, BR), :]
        r = jnp.dot(ab, ebf_scr[...], preferred_element_type=jnp.float32)
        e_blk = e_scr[pl.ds(j * BR, BR), :]
        m_scr[...] += jax.lax.dot_general(
            e_blk, r, (((0,), (0,)), ((), ())),
            preferred_element_type=jnp.float32)

    @pl.when(i == 2 * NB - 1)
    def _final():
        m = m_scr[...]
        nrow = nrow_scr[...]
        ncol = ncol_scr[...]
        ri = jax.lax.broadcasted_iota(jnp.int32, (K, K), 0)
        ci = jax.lax.broadcasted_iota(jnp.int32, (K, K), 1)
        eye = ri == ci
        counts_raw = ncol * nrow - jnp.where(eye, nrow, 0.0)
        counts = jnp.where(counts_raw == 0.0, 1.0, counts_raw)
        p = m / counts
        l1p = jnp.log(p + EPS)
        l0p = jnp.log(1.0 - p + EPS)
        ll = 0.5 * jnp.sum(m * l1p + (counts_raw - m) * l0p)
        p_ref[...] = p
        ll_ref[...] = jnp.full((1, 1), ll, jnp.float32)


def kernel(adj_matrix, block_assignments, block_probs):
    z2 = block_assignments.astype(jnp.int32).reshape(N, 1)

    last = NB - 1
    resp, asg, p, ll = pl.pallas_call(
        _em_body,
        grid=(2 * NB,),
        in_specs=[
            pl.BlockSpec((N, 1), lambda i: (0, 0)),
            pl.BlockSpec((K, K), lambda i: (0, 0)),
            pl.BlockSpec((HR, N), lambda i: (2 * jnp.minimum(i, last), 0)),
            pl.BlockSpec((HR, N), lambda i: (2 * jnp.minimum(i, last) + 1, 0)),
        ],
        out_specs=[
            pl.BlockSpec((BR, K), lambda i: (jnp.minimum(i, last), 0)),
            pl.BlockSpec((BR, 1), lambda i: (jnp.minimum(i, last), 0)),
            pl.BlockSpec((K, K), lambda i: (0, 0)),
            pl.BlockSpec((1, 1), lambda i: (0, 0)),
        ],
        out_shape=[
            jax.ShapeDtypeStruct((N, K), jnp.float32),
            jax.ShapeDtypeStruct((N, 1), jnp.int32),
            jax.ShapeDtypeStruct((K, K), jnp.float32),
            jax.ShapeDtypeStruct((1, 1), jnp.float32),
        ],
        scratch_shapes=[
            pltpu.VMEM((N, K), jnp.float32),
            pltpu.VMEM((N, K), jnp.float32),
            pltpu.VMEM((N, N), jnp.float8_e4m3fn),
            pltpu.VMEM((N, 1), jnp.int32),
            pltpu.VMEM((N, K), jnp.float32),
            pltpu.VMEM((N, K), jnp.float8_e4m3fn),
            pltpu.VMEM((K, K), jnp.float32),
            pltpu.VMEM((1, K), jnp.float32),
            pltpu.VMEM((K, 1), jnp.float32),
        ],
    )(z2, block_probs, adj_matrix, adj_matrix)

    return resp, asg.reshape(N), p, ll.reshape(())


# speedup vs baseline: 1.5790x; 1.0542x over previous
import jax
import jax.numpy as jnp
from jax.experimental import pallas as pl
from jax.experimental.pallas import tpu as pltpu

N = 4096
K = 32
EPS = 1e-10
BR = 512
NB = N // BR
PREC = jax.lax.Precision.HIGHEST


def _em_body(z_ref, bp_ref, adj_ref,
             resp_ref, asg_ref, p_ref, ll_ref,
             l1z_scr, l0z_scr, abf_scr, zn_scr, e_scr, ebf_scr,
             m_scr, nrow_scr, ncol_scr):
    i = pl.program_id(0)

    @pl.when(i == 0)
    def _init_e():
        bp = bp_ref[...]
        l1t = jnp.log(bp + EPS).T
        l0t = jnp.log(1.0 - bp + EPS).T
        kio = jax.lax.broadcasted_iota(jnp.int32, (N, K), 1)
        onehot = (z_ref[...] == kio).astype(jnp.float32)
        l1z_scr[...] = jnp.dot(onehot, l1t,
                               preferred_element_type=jnp.float32, precision=PREC)
        l0z_scr[...] = jnp.dot(onehot, l0t,
                               preferred_element_type=jnp.float32, precision=PREC)

    @pl.when(i < NB)
    def _estep():
        a = adj_ref[...]
        abf_scr[pl.ds(i * BR, BR), :] = a.astype(jnp.float8_e4m3fn)
        logp = (jnp.dot(a, l1z_scr[...], preferred_element_type=jnp.float32)
                + jnp.dot(1.0 - a, l0z_scr[...],
                          preferred_element_type=jnp.float32))
        logp = logp - l0z_scr[pl.ds(i * BR, BR), :]
        m = jnp.max(logp, axis=1, keepdims=True)
        e = jnp.exp(logp - m)
        r = e / jnp.sum(e, axis=1, keepdims=True)
        resp_ref[...] = r
        mx = jnp.max(r, axis=1, keepdims=True)
        kio = jax.lax.broadcasted_iota(jnp.int32, (BR, K), 1)
        asg = jnp.min(jnp.where(r == mx, kio, K), axis=1, keepdims=True)
        asg_ref[...] = asg
        zn_scr[pl.ds(i * BR, BR), :] = asg

    @pl.when(i == NB)
    def _init_m():
        kio = jax.lax.broadcasted_iota(jnp.int32, (N, K), 1)
        e_full = (zn_scr[...] == kio).astype(jnp.float32)
        e_scr[...] = e_full
        ebf_scr[...] = e_full.astype(jnp.float8_e4m3fn)
        nrow_scr[...] = jnp.sum(e_full, axis=0, keepdims=True)
        ones = jnp.ones((N, 1), jnp.float32)
        ncol_scr[...] = jax.lax.dot_general(
            e_full, ones, (((0,), (0,)), ((), ())),
            preferred_element_type=jnp.float32)
        m_scr[...] = jnp.zeros((K, K), jnp.float32)

    @pl.when(i >= NB)
    def _mstep():
        j = i - NB
        ab = abf_scr[pl.ds(j * BR, BR), :]
        r = jnp.dot(ab, ebf_scr[...], preferred_element_type=jnp.float32)
        e_blk = e_scr[pl.ds(j * BR, BR), :]
        m_scr[...] += jax.lax.dot_general(
            e_blk, r, (((0,), (0,)), ((), ())),
            preferred_element_type=jnp.float32)

    @pl.when(i == 2 * NB - 1)
    def _final():
        m = m_scr[...]
        nrow = nrow_scr[...]
        ncol = ncol_scr[...]
        ri = jax.lax.broadcasted_iota(jnp.int32, (K, K), 0)
        ci = jax.lax.broadcasted_iota(jnp.int32, (K, K), 1)
        eye = ri == ci
        counts_raw = ncol * nrow - jnp.where(eye, nrow, 0.0)
        counts = jnp.where(counts_raw == 0.0, 1.0, counts_raw)
        p = m / counts
        l1p = jnp.log(p + EPS)
        l0p = jnp.log(1.0 - p + EPS)
        ll = 0.5 * jnp.sum(m * l1p + (counts_raw - m) * l0p)
        p_ref[...] = p
        ll_ref[...] = jnp.full((1, 1), ll, jnp.float32)


def kernel(adj_matrix, block_assignments, block_probs):
    z2 = block_assignments.astype(jnp.int32).reshape(N, 1)

    last = NB - 1
    resp, asg, p, ll = pl.pallas_call(
        _em_body,
        grid=(2 * NB,),
        in_specs=[
            pl.BlockSpec((N, 1), lambda i: (0, 0)),
            pl.BlockSpec((K, K), lambda i: (0, 0)),
            pl.BlockSpec((BR, N), lambda i: (jnp.minimum(i, last), 0)),
        ],
        out_specs=[
            pl.BlockSpec((BR, K), lambda i: (jnp.minimum(i, last), 0)),
            pl.BlockSpec((BR, 1), lambda i: (jnp.minimum(i, last), 0)),
            pl.BlockSpec((K, K), lambda i: (0, 0)),
            pl.BlockSpec((1, 1), lambda i: (0, 0)),
        ],
        out_shape=[
            jax.ShapeDtypeStruct((N, K), jnp.float32),
            jax.ShapeDtypeStruct((N, 1), jnp.int32),
            jax.ShapeDtypeStruct((K, K), jnp.float32),
            jax.ShapeDtypeStruct((1, 1), jnp.float32),
        ],
        scratch_shapes=[
            pltpu.VMEM((N, K), jnp.float32),
            pltpu.VMEM((N, K), jnp.float32),
            pltpu.VMEM((N, N), jnp.float8_e4m3fn),
            pltpu.VMEM((N, 1), jnp.int32),
            pltpu.VMEM((N, K), jnp.float32),
            pltpu.VMEM((N, K), jnp.float8_e4m3fn),
            pltpu.VMEM((K, K), jnp.float32),
            pltpu.VMEM((1, K), jnp.float32),
            pltpu.VMEM((K, 1), jnp.float32),
        ],
    )(z2, block_probs, adj_matrix)

    return resp, asg.reshape(N), p, ll.reshape(())
